# baseline (speedup 1.0000x reference)
import functools
import math

import numpy as np
import jax
import jax.numpy as jnp
from jax import lax
from jax.experimental import pallas as pl
from jax.experimental.pallas import tpu as pltpu
from jax.experimental.pallas import tpu_sc as plsc

_NUM_EMB = 1000000
_D = 64
_SEQ = 200
_B = 4096

_NW = 32
_BPT = _B // _NW
_NG = _BPT // 16


def _pe_table():
    position = np.arange(_SEQ, dtype=np.float32)[:, None]
    div = np.exp(
        np.arange(0, _D, 2, dtype=np.float32) * (-math.log(10000.0) / _D)
    )
    pe = np.zeros((_SEQ, _D), dtype=np.float32)
    pe[:, 0::2] = np.sin(position * div)
    pe[:, 1::2] = np.cos(position * div)
    return pe


_MESH = plsc.VectorSubcoreMesh(core_axis_name="c", subcore_axis_name="s")


def kernel(data, table):
    pe = jnp.asarray(_pe_table())
    idx_t = data.astype(jnp.int32).T
    t2 = table.reshape(_NUM_EMB // 2, 2 * _D)

    @functools.partial(
        pl.kernel,
        out_type=jax.ShapeDtypeStruct((_SEQ, _D, _B), jnp.float32),
        mesh=_MESH,
        scratch_types=[
            pltpu.VMEM((_SEQ, _BPT), jnp.int32),
            pltpu.VMEM((_SEQ, _D), jnp.float32),
            pltpu.VMEM((2, _BPT), jnp.int32),
            pltpu.VMEM((2, _BPT, 2 * _D), jnp.float32),
            pltpu.VMEM((2, _D, _BPT), jnp.float32),
            pltpu.SemaphoreType.DMA((2,)),
            pltpu.SemaphoreType.DMA((2,)),
            pltpu.SemaphoreType.DMA,
        ],
        compiler_params=pltpu.CompilerParams(
            use_tc_tiling_on_sc=True, needs_layout_passes=False
        ),
    )
    def _emb(idx_hbm, pe_hbm, t2_hbm, out_hbm, idx_v, pe_v, sidx, G, OS,
             gsem, osem, psem):
        wid = lax.axis_index("s") * 2 + lax.axis_index("c")
        b0 = wid * _BPT

        cp_i = pltpu.async_copy(idx_hbm.at[:, pl.ds(b0, _BPT)], idx_v, psem)
        cp_p = pltpu.async_copy(pe_hbm, pe_v, psem)
        cp_i.wait()
        cp_p.wait()

        iota = lax.broadcasted_iota(jnp.int32, (16,), 0)
        riotas = [iota + 16 * g for g in range(_NG)]

        def issue_gather(s, k):
            for g in range(_NG):
                sl = pl.ds(16 * g, 16)
                sidx[k, sl] = lax.shift_right_logical(idx_v[s, sl], 1)
            pltpu.make_async_copy(
                t2_hbm.at[sidx.at[k]], G.at[k], gsem.at[k]
            ).start()

        def wait_gather(k):
            pltpu.make_async_copy(
                t2_hbm.at[sidx.at[k]], G.at[k], gsem.at[k]
            ).wait()

        def out_cp(p, k):
            return pltpu.make_async_copy(
                OS.at[k], out_hbm.at[p, :, pl.ds(b0, _BPT)], osem.at[k]
            )

        def process(p, k):
            wait_gather(k)
            Gk = G.at[k]
            OSk = OS.at[k]
            pars = []
            for g in range(_NG):
                sl = pl.ds(16 * g, 16)
                pars.append(
                    lax.shift_left(
                        lax.bitwise_and(idx_v[p, sl], 1), 6
                    )
                )

            pv = jnp.full((16,), p, dtype=jnp.int32)
            for d in range(_D):
                dv = jnp.full((16,), d, dtype=jnp.int32)
                pev = plsc.load_gather(pe_v, [pv, dv])
                for g in range(_NG):
                    vals = plsc.load_gather(Gk, [riotas[g], dv + pars[g]])
                    OSk[d, pl.ds(16 * g, 16)] = vals + pev

            out_cp(p, k).start()

        issue_gather(0, 0)

        @pl.loop(0, _SEQ // 2)
        def _(t):
            for k in range(2):
                s = 2 * t + k
                if k == 1:
                    @pl.when(t >= 1)
                    def _():
                        out_cp(s - 2, 1).wait()

                    issue_gather(s, 1)
                    process(s - 1, 0)
                else:

                    @pl.when(t >= 1)
                    def _():
                        out_cp(s - 2, 0).wait()
                        issue_gather(s, 0)
                        process(s - 1, 1)

        process(_SEQ - 1, 1)
        out_cp(_SEQ - 2, 0).wait()
        out_cp(_SEQ - 1, 1).wait()

    out = _emb(idx_t, pe, t2)
    return out.transpose(2, 0, 1)


# speedup vs baseline: 1.0756x; 1.0756x over previous
import functools
import math

import numpy as np
import jax
import jax.numpy as jnp
from jax import lax
from jax.experimental import pallas as pl
from jax.experimental.pallas import tpu as pltpu
from jax.experimental.pallas import tpu_sc as plsc

_NUM_EMB = 1000000
_D = 64
_SEQ = 200
_B = 4096

_NW = 32
_BPT = _B // _NW
_NG = _BPT // 16


def _pe_table():
    position = np.arange(_SEQ, dtype=np.float32)[:, None]
    div = np.exp(
        np.arange(0, _D, 2, dtype=np.float32) * (-math.log(10000.0) / _D)
    )
    pe = np.zeros((_SEQ, _D), dtype=np.float32)
    pe[:, 0::2] = np.sin(position * div)
    pe[:, 1::2] = np.cos(position * div)
    return pe


_MESH = plsc.VectorSubcoreMesh(core_axis_name="c", subcore_axis_name="s")


def kernel(data, table):
    pe_np = np.zeros((_SEQ, 2 * _D), dtype=np.float32)
    pe_np[:, : _D] = _pe_table()
    pe = jnp.asarray(pe_np)
    idx_t = data.astype(jnp.int32).T
    t2 = table.reshape(_NUM_EMB // 2, 2 * _D)

    @functools.partial(
        pl.kernel,
        out_type=jax.ShapeDtypeStruct((_SEQ, _D, _B), jnp.float32),
        mesh=_MESH,
        scratch_types=[
            pltpu.VMEM((_SEQ, _BPT), jnp.int32),
            pltpu.VMEM((4, 2 * _D), jnp.float32),
            pltpu.VMEM((4, _BPT), jnp.int32),
            pltpu.VMEM((4, _BPT, 2 * _D), jnp.float32),
            pltpu.VMEM((2, _D, _BPT), jnp.float32),
            pltpu.SemaphoreType.DMA((4,)),
            pltpu.SemaphoreType.DMA((2,)),
            pltpu.SemaphoreType.DMA,
        ],
        compiler_params=pltpu.CompilerParams(
            use_tc_tiling_on_sc=True, needs_layout_passes=False
        ),
    )
    def _emb(idx_hbm, pe_hbm, t2_hbm, out_hbm, idx_v, pe_b, sidx, G, OS,
             gsem, osem, psem):
        wid = lax.axis_index("s") * 2 + lax.axis_index("c")
        b0 = wid * _BPT

        pltpu.async_copy(idx_hbm.at[:, pl.ds(b0, _BPT)], idx_v, psem).wait()

        iota = lax.broadcasted_iota(jnp.int32, (16,), 0)
        riotas = [iota + 16 * g for g in range(_NG)]

        def issue_gather(s, k):
            for g in range(_NG):
                sl = pl.ds(16 * g, 16)
                sidx[k, sl] = lax.shift_right_logical(idx_v[s, sl], 1)
            pltpu.make_async_copy(
                t2_hbm.at[sidx.at[k]], G.at[k], gsem.at[k]
            ).start()
            pltpu.make_async_copy(
                pe_hbm.at[s], pe_b.at[k], gsem.at[k]
            ).start()

        def wait_gather(k):
            pltpu.make_async_copy(
                t2_hbm.at[sidx.at[k]], G.at[k], gsem.at[k]
            ).wait()
            pltpu.make_async_copy(
                pe_hbm.at[0], pe_b.at[k], gsem.at[k]
            ).wait()

        def out_cp(p, k):
            return pltpu.make_async_copy(
                OS.at[k], out_hbm.at[p, :, pl.ds(b0, _BPT)], osem.at[k]
            )

        def process(p, k, ko):
            wait_gather(k)
            Gk = G.at[k]
            OSk = OS.at[ko]
            pars = []
            for g in range(_NG):
                sl = pl.ds(16 * g, 16)
                pars.append(
                    lax.shift_left(
                        lax.bitwise_and(idx_v[p, sl], 1), 6
                    )
                )

            peb = pe_b.at[k]

            @pl.loop(0, _D, step=4)
            def _(d0):
                for dd in range(4):
                    d = d0 + dd
                    dv = jnp.full((16,), d, dtype=jnp.int32)
                    pev = plsc.load_gather(peb, [dv])
                    for g in range(_NG):
                        vals = plsc.load_gather(Gk, [riotas[g], dv + pars[g]])
                        OSk[d, pl.ds(16 * g, 16)] = vals + pev

            out_cp(p, ko).start()

        for j in range(3):
            issue_gather(j, j)

        @pl.loop(0, _SEQ // 4)
        def _(t):
            for k in range(4):
                s = 4 * t + k
                kg = (k + 3) % 4
                if k == 0:
                    issue_gather(s + 3, kg)
                else:

                    @pl.when(t <= _SEQ // 4 - 2)
                    def _():
                        issue_gather(s + 3, kg)

                if k >= 2:
                    out_cp(s - 2, k % 2).wait()
                else:

                    @pl.when(t >= 1)
                    def _():
                        out_cp(s - 2, k % 2).wait()

                process(s, k, k % 2)

        out_cp(_SEQ - 2, 0).wait()
        out_cp(_SEQ - 1, 1).wait()

    out = _emb(idx_t, pe, t2)
    return out.transpose(2, 0, 1)


# speedup vs baseline: 1.5997x; 1.4873x over previous
import functools
import math

import numpy as np
import jax
import jax.numpy as jnp
from jax import lax
from jax.experimental import pallas as pl
from jax.experimental.pallas import tpu as pltpu
from jax.experimental.pallas import tpu_sc as plsc

_NUM_EMB = 1000000
_D = 64
_SEQ = 200
_B = 4096

_NW = 32
_BPT = _B // _NW
_NG = _BPT // 16


def _pe_table():
    position = np.arange(_SEQ, dtype=np.float32)[:, None]
    div = np.exp(
        np.arange(0, _D, 2, dtype=np.float32) * (-math.log(10000.0) / _D)
    )
    pe = np.zeros((_SEQ, _D), dtype=np.float32)
    pe[:, 0::2] = np.sin(position * div)
    pe[:, 1::2] = np.cos(position * div)
    return pe


_MESH = plsc.VectorSubcoreMesh(core_axis_name="c", subcore_axis_name="s")


def kernel(data, table):
    pe_np = np.zeros((_SEQ, 2 * _D), dtype=np.float32)
    pe_np[:, : _D] = _pe_table()
    pe = jnp.asarray(pe_np)
    idx_t = data.astype(jnp.int32).T
    t2 = table.reshape(_NUM_EMB // 2, 2 * _D)

    @functools.partial(
        pl.kernel,
        out_type=jax.ShapeDtypeStruct((_SEQ, _D, _B), jnp.float32),
        mesh=_MESH,
        scratch_types=[
            pltpu.VMEM((_SEQ, _BPT), jnp.int32),
            pltpu.VMEM((4, 2 * _D), jnp.float32),
            pltpu.VMEM((4, _BPT), jnp.int32),
            pltpu.VMEM((4, _BPT, 2 * _D), jnp.float32),
            pltpu.VMEM((2, _D, _BPT), jnp.float32),
            pltpu.SemaphoreType.DMA((4,)),
            pltpu.SemaphoreType.DMA((2,)),
            pltpu.SemaphoreType.DMA,
        ],
        compiler_params=pltpu.CompilerParams(
            use_tc_tiling_on_sc=True, needs_layout_passes=False
        ),
    )
    def _emb(idx_hbm, pe_hbm, t2_hbm, out_hbm, idx_v, pe_b, sidx, G, OS,
             gsem, osem, psem):
        wid = lax.axis_index("s") * 2 + lax.axis_index("c")
        b0 = wid * _BPT

        pltpu.async_copy(idx_hbm.at[:, pl.ds(b0, _BPT)], idx_v, psem).wait()

        iota = lax.broadcasted_iota(jnp.int32, (16,), 0)
        riotas = [iota + 16 * g for g in range(_NG)]

        def issue_gather(s, k):
            for g in range(_NG):
                sl = pl.ds(16 * g, 16)
                sidx[k, sl] = lax.shift_right_logical(idx_v[s, sl], 1)
            pltpu.make_async_copy(
                t2_hbm.at[sidx.at[k]], G.at[k], gsem.at[k]
            ).start()
            pltpu.make_async_copy(
                pe_hbm.at[s], pe_b.at[k], gsem.at[k]
            ).start()

        def wait_gather(k):
            pltpu.make_async_copy(
                t2_hbm.at[sidx.at[k]], G.at[k], gsem.at[k]
            ).wait()
            pltpu.make_async_copy(
                pe_hbm.at[0], pe_b.at[k], gsem.at[k]
            ).wait()

        def out_cp(p, k):
            return pltpu.make_async_copy(
                OS.at[k], out_hbm.at[p, :, pl.ds(b0, _BPT)], osem.at[k]
            )

        def process(p, k, ko):
            wait_gather(k)
            Gk = G.at[k]
            OSk = OS.at[ko]
            pars = []
            for g in range(_NG):
                sl = pl.ds(16 * g, 16)
                pars.append(
                    lax.shift_left(
                        lax.bitwise_and(idx_v[p, sl], 1), 6
                    )
                )

            peb = pe_b.at[k]

            @plsc.parallel_loop(0, _D, step=4, unroll=2)
            def _(d0):
                for dd in range(4):
                    d = d0 + dd
                    dv = jnp.full((16,), d, dtype=jnp.int32)
                    pev = plsc.load_gather(peb, [dv])
                    for g in range(_NG):
                        vals = plsc.load_gather(Gk, [riotas[g], dv + pars[g]])
                        OSk[d, pl.ds(16 * g, 16)] = vals + pev

            out_cp(p, ko).start()

        for j in range(3):
            issue_gather(j, j)

        @pl.loop(0, _SEQ // 4)
        def _(t):
            for k in range(4):
                s = 4 * t + k
                kg = (k + 3) % 4
                if k == 0:
                    issue_gather(s + 3, kg)
                else:

                    @pl.when(t <= _SEQ // 4 - 2)
                    def _():
                        issue_gather(s + 3, kg)

                if k >= 2:
                    out_cp(s - 2, k % 2).wait()
                else:

                    @pl.when(t >= 1)
                    def _():
                        out_cp(s - 2, k % 2).wait()

                process(s, k, k % 2)

        out_cp(_SEQ - 2, 0).wait()
        out_cp(_SEQ - 1, 1).wait()

    out = _emb(idx_t, pe, t2)
    return out.transpose(2, 0, 1)


# speedup vs baseline: 1.6096x; 1.0061x over previous
import functools
import math

import numpy as np
import jax
import jax.numpy as jnp
from jax import lax
from jax.experimental import pallas as pl
from jax.experimental.pallas import tpu as pltpu
from jax.experimental.pallas import tpu_sc as plsc

_NUM_EMB = 1000000
_D = 64
_SEQ = 200
_B = 4096

_NW = 32
_BPT = _B // _NW
_NG = _BPT // 16


def _pe_table():
    position = np.arange(_SEQ, dtype=np.float32)[:, None]
    div = np.exp(
        np.arange(0, _D, 2, dtype=np.float32) * (-math.log(10000.0) / _D)
    )
    pe = np.zeros((_SEQ, _D), dtype=np.float32)
    pe[:, 0::2] = np.sin(position * div)
    pe[:, 1::2] = np.cos(position * div)
    return pe


_MESH = plsc.VectorSubcoreMesh(core_axis_name="c", subcore_axis_name="s")


def kernel(data, table):
    pe_np = np.zeros((_SEQ, 2 * _D), dtype=np.float32)
    pe_np[:, : _D] = _pe_table()
    pe = jnp.asarray(pe_np)
    idx_t = data.astype(jnp.int32).T
    t2 = table.reshape(_NUM_EMB // 2, 2 * _D)

    @functools.partial(
        pl.kernel,
        out_type=jax.ShapeDtypeStruct((_SEQ, _D, _B), jnp.float32),
        mesh=_MESH,
        scratch_types=[
            pltpu.VMEM((_SEQ, _BPT), jnp.int32),
            pltpu.VMEM((4, 2 * _D), jnp.float32),
            pltpu.VMEM((4, _BPT), jnp.int32),
            pltpu.VMEM((4, _BPT, 2 * _D), jnp.float32),
            pltpu.VMEM((2, _D, _BPT), jnp.float32),
            pltpu.SemaphoreType.DMA((4,)),
            pltpu.SemaphoreType.DMA((2,)),
            pltpu.SemaphoreType.DMA,
        ],
        compiler_params=pltpu.CompilerParams(
            use_tc_tiling_on_sc=True, needs_layout_passes=False
        ),
    )
    def _emb(idx_hbm, pe_hbm, t2_hbm, out_hbm, idx_v, pe_b, sidx, G, OS,
             gsem, osem, psem):
        wid = lax.axis_index("s") * 2 + lax.axis_index("c")
        b0 = wid * _BPT

        pltpu.async_copy(idx_hbm.at[:, pl.ds(b0, _BPT)], idx_v, psem).wait()

        iota = lax.broadcasted_iota(jnp.int32, (16,), 0)
        riotas = [iota + 16 * g for g in range(_NG)]

        def issue_gather(s, k):
            for g in range(_NG):
                sl = pl.ds(16 * g, 16)
                sidx[k, sl] = lax.shift_right_logical(idx_v[s, sl], 1)
            pltpu.make_async_copy(
                t2_hbm.at[sidx.at[k]], G.at[k], gsem.at[k]
            ).start()
            pltpu.make_async_copy(
                pe_hbm.at[s], pe_b.at[k], gsem.at[k]
            ).start()

        def wait_gather(k):
            pltpu.make_async_copy(
                t2_hbm.at[sidx.at[k]], G.at[k], gsem.at[k]
            ).wait()
            pltpu.make_async_copy(
                pe_hbm.at[0], pe_b.at[k], gsem.at[k]
            ).wait()

        def out_cp(p, k):
            return pltpu.make_async_copy(
                OS.at[k], out_hbm.at[p, :, pl.ds(b0, _BPT)], osem.at[k]
            )

        def process(p, k, ko):
            wait_gather(k)
            Gk = G.at[k]
            OSk = OS.at[ko]
            pars = []
            for g in range(_NG):
                sl = pl.ds(16 * g, 16)
                pars.append(
                    lax.shift_left(
                        lax.bitwise_and(idx_v[p, sl], 1), 6
                    )
                )

            peb = pe_b.at[k]

            @plsc.parallel_loop(0, _D, step=4, unroll=4)
            def _(d0):
                for dd in range(4):
                    d = d0 + dd
                    dv = jnp.full((16,), d, dtype=jnp.int32)
                    pev = plsc.load_gather(peb, [dv])
                    for g in range(_NG):
                        vals = plsc.load_gather(Gk, [riotas[g], dv + pars[g]])
                        OSk[d, pl.ds(16 * g, 16)] = vals + pev

            out_cp(p, ko).start()

        for j in range(3):
            issue_gather(j, j)

        @pl.loop(0, _SEQ // 4)
        def _(t):
            for k in range(4):
                s = 4 * t + k
                kg = (k + 3) % 4
                if k == 0:
                    issue_gather(s + 3, kg)
                else:

                    @pl.when(t <= _SEQ // 4 - 2)
                    def _():
                        issue_gather(s + 3, kg)

                if k >= 2:
                    out_cp(s - 2, k % 2).wait()
                else:

                    @pl.when(t >= 1)
                    def _():
                        out_cp(s - 2, k % 2).wait()

                process(s, k, k % 2)

        out_cp(_SEQ - 2, 0).wait()
        out_cp(_SEQ - 1, 1).wait()

    out = _emb(idx_t, pe, t2)
    return out.transpose(2, 0, 1)
